# Initial kernel scaffold; baseline (speedup 1.0000x reference)
import jax
import jax.numpy as jnp
from jax.experimental import pallas as pl


def kernel(h, section_feature):
    raise NotImplementedError("write your pallas kernel here")


# speedup vs baseline: 1.7706x; 1.7706x over previous
import functools

import jax
import jax.numpy as jnp
from jax import lax
from jax.experimental import pallas as pl
from jax.experimental.pallas import tpu as pltpu
from jax.experimental.pallas import tpu_sc as plsc

B = 4
N = 50000
D = 128
K = 8192

NT = 16
NC = 2
NPAD = 51200
CHUNK = NPAD // NT
NVREG = CHUNK // 16
RADIX = 256
NPASS = 4
NSCAT = CHUNK // 128
KPT = K // NT
SENT = 0x3FFFFFFF

BLKN = 2000


def _score_body(h_ref, sf_ref, out_ref):
  w = lax.dot_general(
      sf_ref[0], h_ref[0],
      dimension_numbers=(((1,), (1,)), ((), ())),
      preferred_element_type=jnp.float32,
  )
  score = pl.reciprocal(1.0 + jnp.exp(-w), approx=False)
  skey = SENT - lax.bitcast_convert_type(score, jnp.int32)
  out_ref[0] = skey


def _scores(h, section_feature):
  grid = (B, N // BLKN)
  out = pl.pallas_call(
      _score_body,
      grid=grid,
      in_specs=[
          pl.BlockSpec((1, BLKN, D), lambda b, j: (b, j, 0)),
          pl.BlockSpec((1, 1, D), lambda b, j: (b, 0, 0)),
      ],
      out_specs=pl.BlockSpec((1, 1, BLKN), lambda b, j: (b * (N // BLKN) + j, 0, 0)),
      out_shape=jax.ShapeDtypeStruct((B * (N // BLKN), 1, BLKN), jnp.int32),
  )(h, section_feature)
  return out.reshape(B, N)


def _sc_body(skey_hbm, h_hbm, out_hbm,
             akey, aidx, bkey, bidx, grid_sh,
             keyv, idxv, lrankv, posv, histv, basev, gridv,
             tmpa, tmpb, tmpc, gkeyv, gidxv, gidx2, rowsv, sem):
  c = lax.axis_index("c")
  t = lax.axis_index("s")
  iota16 = lax.iota(jnp.int32, 16)
  lane0 = iota16 == 0
  lane15 = iota16 == 15
  idxm1 = jnp.maximum(iota16 - 1, 0)
  idxp1 = jnp.minimum(iota16 + 1, 15)
  splat15 = jnp.full((16,), 15, jnp.int32)
  splat_t = jnp.full((16,), t, jnp.int32)
  iota_r = iota16 * RADIX
  zeros16 = jnp.zeros((16,), jnp.int32)

  for r in range(2):
    b = c * 2 + r
    for p in range(NPASS):
      shift = 8 * p
      if p % 2 == 0:
        dst_key, dst_idx = akey, aidx
        src_key, src_idx = bkey, bidx
      else:
        dst_key, dst_idx = bkey, bidx
        src_key, src_idx = akey, aidx
      if p == 0:
        pltpu.sync_copy(skey_hbm.at[pl.ds(b * NPAD + t * CHUNK, CHUNK)], keyv)

        def genidx(j, carry):
          idxv[pl.ds(j * 16, 16)] = t * CHUNK + j * 16 + iota16
          return carry

        lax.fori_loop(0, NVREG, genidx, 0)
      else:
        pltpu.sync_copy(src_key.at[pl.ds(t * CHUNK, CHUNK)], keyv)
        pltpu.sync_copy(src_idx.at[pl.ds(t * CHUNK, CHUNK)], idxv)
      for z in range(RADIX // 16):
        histv[pl.ds(z * 16, 16)] = zeros16

      def step_a(j, carry):
        k = keyv[pl.ds(j * 16, 16)]
        d = (k >> shift) & (RADIX - 1)
        ds, lanes = plsc.sort_key_val(d, iota16)
        tmpa[pl.ds(0, 16)] = ds
        prev = plsc.load_gather(tmpa, [idxm1])
        is_start = lane0 | (ds != prev)
        startpos = plsc.cummax(jnp.where(is_start, iota16, 0))
        runrank = iota16 - startpos
        cur = plsc.load_gather(histv, [ds])
        lrank_s = cur + runrank
        tmpb[pl.ds(0, 16)] = jnp.where(is_start, 1, 0)
        nxt = plsc.load_gather(tmpb, [idxp1])
        is_last = lane15 | (nxt == 1)
        plsc.store_scatter(histv, [ds], lrank_s + 1, mask=is_last)
        plsc.store_scatter(tmpc, [lanes], lrank_s)
        lrankv[pl.ds(j * 16, 16)] = tmpc[pl.ds(0, 16)]
        return carry

      lax.fori_loop(0, NVREG, step_a, 0)
      pltpu.sync_copy(histv, grid_sh.at[pl.ds(t * RADIX, RADIX)])
      plsc.subcore_barrier()
      pltpu.sync_copy(grid_sh, gridv)

      def step_s(d, run):
        cvec = plsc.load_gather(gridv, [iota_r + d])
        inc = plsc.cumsum(cvec)
        tmpa[pl.ds(0, 16)] = inc - cvec
        tmpb[pl.ds(0, 16)] = inc
        myexcl = plsc.load_gather(tmpa, [splat_t])
        tot = plsc.load_gather(tmpb, [splat15])
        mybase = run + myexcl
        plsc.store_scatter(basev, [jnp.full((16,), d, jnp.int32)], mybase,
                           mask=lane0)
        return run + tot

      lax.fori_loop(0, RADIX, step_s, zeros16)

      def step_b(j, carry):
        k = keyv[pl.ds(j * 16, 16)]
        d = (k >> shift) & (RADIX - 1)
        lr = lrankv[pl.ds(j * 16, 16)]
        pos = plsc.load_gather(basev, [d]) + lr
        posv[j // 8, pl.ds((j % 8) * 16, 16)] = pos
        return carry

      lax.fori_loop(0, NVREG, step_b, 0)
      for q in range(NSCAT):
        d1 = pltpu.async_copy(keyv.at[pl.ds(q * 128, 128)],
                              dst_key.at[posv.at[q]], sem)
        d2 = pltpu.async_copy(idxv.at[pl.ds(q * 128, 128)],
                              dst_idx.at[posv.at[q]], sem)
        d1.wait()
        d2.wait()
      plsc.subcore_barrier()

    pltpu.sync_copy(bkey.at[pl.ds(t * KPT, KPT)], gkeyv)
    pltpu.sync_copy(bidx.at[pl.ds(t * KPT, KPT)], gidxv)
    base_row = b * N

    def adj(j, carry):
      v = gidxv[pl.ds(j * 16, 16)] + base_row
      gidx2[j // 8, pl.ds((j % 8) * 16, 16)] = v
      return carry

    lax.fori_loop(0, KPT // 16, adj, 0)
    for g in range(KPT // 128):
      pltpu.async_copy(h_hbm.at[gidx2.at[g]],
                       rowsv.at[pl.ds(g * 128, 128)], sem).wait()

    def scale(j, carry):
      kv = plsc.load_gather(gkeyv, [jnp.full((16,), j, jnp.int32)])
      sc = plsc.bitcast(jnp.full((16,), SENT, jnp.int32) - kv, jnp.float32)
      for q in range(8):
        rowsv[j, pl.ds(q * 16, 16)] = rowsv[j, pl.ds(q * 16, 16)] * sc
      return carry

    lax.fori_loop(0, KPT, scale, 0)
    pltpu.sync_copy(rowsv, out_hbm.at[pl.ds(b * K + t * KPT, KPT)])


@functools.partial(
    pl.kernel,
    out_type=jax.ShapeDtypeStruct((B * K, D), jnp.float32),
    mesh=plsc.VectorSubcoreMesh(core_axis_name="c", subcore_axis_name="s",
                                num_cores=NC),
    compiler_params=pltpu.CompilerParams(needs_layout_passes=False),
    scratch_types=[
        pltpu.VMEM_SHARED((NPAD,), jnp.int32),
        pltpu.VMEM_SHARED((NPAD,), jnp.int32),
        pltpu.VMEM_SHARED((NPAD,), jnp.int32),
        pltpu.VMEM_SHARED((NPAD,), jnp.int32),
        pltpu.VMEM_SHARED((NT * RADIX,), jnp.int32),
        pltpu.VMEM((CHUNK,), jnp.int32),
        pltpu.VMEM((CHUNK,), jnp.int32),
        pltpu.VMEM((CHUNK,), jnp.int32),
        pltpu.VMEM((NSCAT, 128), jnp.int32),
        pltpu.VMEM((RADIX,), jnp.int32),
        pltpu.VMEM((RADIX,), jnp.int32),
        pltpu.VMEM((NT * RADIX,), jnp.int32),
        pltpu.VMEM((16,), jnp.int32),
        pltpu.VMEM((16,), jnp.int32),
        pltpu.VMEM((16,), jnp.int32),
        pltpu.VMEM((KPT,), jnp.int32),
        pltpu.VMEM((KPT,), jnp.int32),
        pltpu.VMEM((KPT // 128, 128), jnp.int32),
        pltpu.VMEM((KPT, D), jnp.float32),
        pltpu.SemaphoreType.DMA,
    ],
)
def _sc_topk(skey_hbm, h_hbm, out_hbm, *rest):
  _sc_body(skey_hbm, h_hbm, out_hbm, *rest)


def kernel(h, section_feature):
  skey = _scores(h, section_feature)
  skey = jnp.concatenate(
      [skey, jnp.full((B, NPAD - N), SENT, jnp.int32)], axis=1)
  out = _sc_topk(skey.reshape(B * NPAD), h.reshape(B * N, D))
  return out.reshape(B, K, D)


# speedup vs baseline: 2.1246x; 1.1999x over previous
import functools

import jax
import jax.numpy as jnp
from jax import lax
from jax.experimental import pallas as pl
from jax.experimental.pallas import tpu as pltpu
from jax.experimental.pallas import tpu_sc as plsc

B = 4
N = 50000
D = 128
K = 8192

NT = 16
NC = 2
NPAD = 51200
CHUNK = NPAD // NT
NVREG = CHUNK // 16
RADIX = 256
NPASS = 4
NSCAT = CHUNK // 128
KPT = K // NT
SENT = 0x3FFFFFFF

BLKN = 5000


def _score_body(h_ref, sf_ref, out_ref):
  w = lax.dot_general(
      sf_ref[0], h_ref[0],
      dimension_numbers=(((1,), (1,)), ((), ())),
      preferred_element_type=jnp.float32,
  )
  score = pl.reciprocal(1.0 + jnp.exp(-w), approx=False)
  skey = SENT - lax.bitcast_convert_type(score, jnp.int32)
  out_ref[0] = skey


def _scores(h, section_feature):
  grid = (B, N // BLKN)
  out = pl.pallas_call(
      _score_body,
      grid=grid,
      in_specs=[
          pl.BlockSpec((1, BLKN, D), lambda b, j: (b, j, 0)),
          pl.BlockSpec((1, 1, D), lambda b, j: (b, 0, 0)),
      ],
      out_specs=pl.BlockSpec((1, 1, BLKN), lambda b, j: (b * (N // BLKN) + j, 0, 0)),
      out_shape=jax.ShapeDtypeStruct((B * (N // BLKN), 1, BLKN), jnp.int32),
  )(h, section_feature)
  return out.reshape(B, N)


def _sc_body(skey_hbm, h_hbm, out_hbm,
             akey, aidx, bkey, bidx, grid_sh,
             keyv, idxv, lrankv, posv, histv, basev, gridv,
             tmpa, tmpb, tmpc, gkeyv, gidxv, gidx2, rowsv, sem):
  c = lax.axis_index("c")
  t = lax.axis_index("s")
  iota16 = lax.iota(jnp.int32, 16)
  lane0 = iota16 == 0
  lane15 = iota16 == 15
  idxm1 = jnp.maximum(iota16 - 1, 0)
  idxp1 = jnp.minimum(iota16 + 1, 15)
  splat15 = jnp.full((16,), 15, jnp.int32)
  splat_t = jnp.full((16,), t, jnp.int32)
  iota_r = iota16 * RADIX
  zeros16 = jnp.zeros((16,), jnp.int32)

  for r in range(2):
    b = c * 2 + r
    for p in range(NPASS):
      shift = 8 * p
      if p % 2 == 0:
        dst_key, dst_idx = akey, aidx
        src_key, src_idx = bkey, bidx
      else:
        dst_key, dst_idx = bkey, bidx
        src_key, src_idx = akey, aidx
      if p == 0:
        pltpu.sync_copy(skey_hbm.at[pl.ds(b * NPAD + t * CHUNK, CHUNK)], keyv)

        def genidx(j, carry):
          idxv[pl.ds(j * 16, 16)] = t * CHUNK + j * 16 + iota16
          return carry

        lax.fori_loop(0, NVREG, genidx, 0)
      else:
        pltpu.sync_copy(src_key.at[pl.ds(t * CHUNK, CHUNK)], keyv)
        pltpu.sync_copy(src_idx.at[pl.ds(t * CHUNK, CHUNK)], idxv)
      for z in range(RADIX // 16):
        histv[pl.ds(z * 16, 16)] = zeros16

      def step_a(j, carry):
        k = keyv[pl.ds(j * 16, 16)]
        d = (k >> shift) & (RADIX - 1)
        ds, lanes = plsc.sort_key_val(d, iota16)
        tmpa[pl.ds(0, 16)] = ds
        prev = plsc.load_gather(tmpa, [idxm1])
        is_start = lane0 | (ds != prev)
        startpos = plsc.cummax(jnp.where(is_start, iota16, 0))
        runrank = iota16 - startpos
        cur = plsc.load_gather(histv, [ds])
        lrank_s = cur + runrank
        tmpb[pl.ds(0, 16)] = jnp.where(is_start, 1, 0)
        nxt = plsc.load_gather(tmpb, [idxp1])
        is_last = lane15 | (nxt == 1)
        plsc.store_scatter(histv, [ds], lrank_s + 1, mask=is_last)
        plsc.store_scatter(tmpc, [lanes], lrank_s)
        lrankv[pl.ds(j * 16, 16)] = tmpc[pl.ds(0, 16)]
        return carry

      lax.fori_loop(0, NVREG, step_a, 0)
      pltpu.sync_copy(histv, grid_sh.at[pl.ds(t * RADIX, RADIX)])
      plsc.subcore_barrier()
      pltpu.sync_copy(grid_sh, gridv)

      def step_s(d, run):
        cvec = plsc.load_gather(gridv, [iota_r + d])
        inc = plsc.cumsum(cvec)
        tmpa[pl.ds(0, 16)] = inc - cvec
        tmpb[pl.ds(0, 16)] = inc
        myexcl = plsc.load_gather(tmpa, [splat_t])
        tot = plsc.load_gather(tmpb, [splat15])
        mybase = run + myexcl
        plsc.store_scatter(basev, [jnp.full((16,), d, jnp.int32)], mybase,
                           mask=lane0)
        return run + tot

      lax.fori_loop(0, RADIX, step_s, zeros16)

      def step_b(j, carry):
        k = keyv[pl.ds(j * 16, 16)]
        d = (k >> shift) & (RADIX - 1)
        lr = lrankv[pl.ds(j * 16, 16)]
        pos = plsc.load_gather(basev, [d]) + lr
        posv[j // 8, pl.ds((j % 8) * 16, 16)] = pos
        return carry

      lax.fori_loop(0, NVREG, step_b, 0)
      pend = []
      for q in range(NSCAT):
        if p != NPASS - 1:
          pend.append(pltpu.async_copy(keyv.at[pl.ds(q * 128, 128)],
                                       dst_key.at[posv.at[q]], sem))
        pend.append(pltpu.async_copy(idxv.at[pl.ds(q * 128, 128)],
                                     dst_idx.at[posv.at[q]], sem))
      for dsc in pend:
        dsc.wait()
      plsc.subcore_barrier()

    pltpu.sync_copy(bidx.at[pl.ds(t * KPT, KPT)], gidxv)
    base_row = b * N
    base_key = b * NPAD

    def adj(j, carry):
      v = gidxv[pl.ds(j * 16, 16)] + base_row
      gidx2[j // 8, pl.ds((j % 8) * 16, 16)] = v
      return carry

    lax.fori_loop(0, KPT // 16, adj, 0)
    pend = [
        pltpu.async_copy(h_hbm.at[gidx2.at[g]],
                         rowsv.at[pl.ds(g * 128, 128)], sem)
        for g in range(KPT // 128)
    ]
    for dsc in pend:
      dsc.wait()

    def adjk(j, carry):
      v = gidxv[pl.ds(j * 16, 16)] + base_key
      gidx2[j // 8, pl.ds((j % 8) * 16, 16)] = v
      return carry

    lax.fori_loop(0, KPT // 16, adjk, 0)
    pend = [
        pltpu.async_copy(skey_hbm.at[gidx2.at[g]],
                         gkeyv.at[pl.ds(g * 128, 128)], sem)
        for g in range(KPT // 128)
    ]
    for dsc in pend:
      dsc.wait()

    def scale(j, carry):
      kv = plsc.load_gather(gkeyv, [jnp.full((16,), j, jnp.int32)])
      sc = plsc.bitcast(jnp.full((16,), SENT, jnp.int32) - kv, jnp.float32)
      for q in range(8):
        rowsv[j, pl.ds(q * 16, 16)] = rowsv[j, pl.ds(q * 16, 16)] * sc
      return carry

    lax.fori_loop(0, KPT, scale, 0)
    pltpu.sync_copy(rowsv, out_hbm.at[pl.ds(b * K + t * KPT, KPT)])


@functools.partial(
    pl.kernel,
    out_type=jax.ShapeDtypeStruct((B * K, D), jnp.float32),
    mesh=plsc.VectorSubcoreMesh(core_axis_name="c", subcore_axis_name="s",
                                num_cores=NC),
    compiler_params=pltpu.CompilerParams(needs_layout_passes=False),
    scratch_types=[
        pltpu.VMEM_SHARED((NPAD,), jnp.int32),
        pltpu.VMEM_SHARED((NPAD,), jnp.int32),
        pltpu.VMEM_SHARED((NPAD,), jnp.int32),
        pltpu.VMEM_SHARED((NPAD,), jnp.int32),
        pltpu.VMEM_SHARED((NT * RADIX,), jnp.int32),
        pltpu.VMEM((CHUNK,), jnp.int32),
        pltpu.VMEM((CHUNK,), jnp.int32),
        pltpu.VMEM((CHUNK,), jnp.int32),
        pltpu.VMEM((NSCAT, 128), jnp.int32),
        pltpu.VMEM((RADIX,), jnp.int32),
        pltpu.VMEM((RADIX,), jnp.int32),
        pltpu.VMEM((NT * RADIX,), jnp.int32),
        pltpu.VMEM((16,), jnp.int32),
        pltpu.VMEM((16,), jnp.int32),
        pltpu.VMEM((16,), jnp.int32),
        pltpu.VMEM((KPT,), jnp.int32),
        pltpu.VMEM((KPT,), jnp.int32),
        pltpu.VMEM((KPT // 128, 128), jnp.int32),
        pltpu.VMEM((KPT, D), jnp.float32),
        pltpu.SemaphoreType.DMA,
    ],
)
def _sc_topk(skey_hbm, h_hbm, out_hbm, *rest):
  _sc_body(skey_hbm, h_hbm, out_hbm, *rest)


def kernel(h, section_feature):
  skey = _scores(h, section_feature)
  skey = jnp.concatenate(
      [skey, jnp.full((B, NPAD - N), SENT, jnp.int32)], axis=1)
  out = _sc_topk(skey.reshape(B * NPAD), h.reshape(B * N, D))
  return out.reshape(B, K, D)


# speedup vs baseline: 2.7883x; 1.3124x over previous
import functools

import jax
import jax.numpy as jnp
from jax import lax
from jax.experimental import pallas as pl
from jax.experimental.pallas import tpu as pltpu
from jax.experimental.pallas import tpu_sc as plsc

B = 4
N = 50000
D = 128
K = 8192

NT = 16
NC = 2
NPAD = 51200
CHUNK = NPAD // NT
NVREG = CHUNK // 16
RADIX = 1024
NPASS = 3
DGT = RADIX // NT
NSCAT = CHUNK // 128
KPT = K // NT
SENT = 0x3FFFFFFF

BLKN = 5000


def _score_body(h_ref, sf_ref, out_ref):
  w = lax.dot_general(
      sf_ref[0], h_ref[0],
      dimension_numbers=(((1,), (1,)), ((), ())),
      preferred_element_type=jnp.float32,
  )
  score = pl.reciprocal(1.0 + jnp.exp(-w), approx=False)
  skey = SENT - lax.bitcast_convert_type(score, jnp.int32)
  out_ref[0] = skey


def _scores(h, section_feature):
  grid = (B, N // BLKN)
  out = pl.pallas_call(
      _score_body,
      grid=grid,
      in_specs=[
          pl.BlockSpec((1, BLKN, D), lambda b, j: (b, j, 0)),
          pl.BlockSpec((1, 1, D), lambda b, j: (b, 0, 0)),
      ],
      out_specs=pl.BlockSpec((1, 1, BLKN), lambda b, j: (b * (N // BLKN) + j, 0, 0)),
      out_shape=jax.ShapeDtypeStruct((B * (N // BLKN), 1, BLKN), jnp.int32),
  )(h, section_feature)
  return out.reshape(B, N)


def _sc_body(skey_hbm, h_hbm, out_hbm,
             akey, aidx, bkey, bidx, grid_sh, base_sh, btot_sh,
             keyv, idxv, lrankv, posv, histv, basev, scanv,
             pubix2, posb2, btotv,
             tmpa, tmpb, tmpc, gkeyv, gidxv, gidx2, rowsv, sem):
  c = lax.axis_index("c")
  t = lax.axis_index("s")
  iota16 = lax.iota(jnp.int32, 16)
  lane0 = iota16 == 0
  lane15 = iota16 == 15
  idxm1 = jnp.maximum(iota16 - 1, 0)
  idxp1 = jnp.minimum(iota16 + 1, 15)
  splat15 = jnp.full((16,), 15, jnp.int32)
  splat_t = jnp.full((16,), t, jnp.int32)
  iota_r = iota16 * RADIX
  zeros16 = jnp.zeros((16,), jnp.int32)

  def initix(j, carry):
    pubix2[j // 8, pl.ds((j % 8) * 16, 16)] = (j * 16 + iota16) * NT + t
    return carry

  lax.fori_loop(0, RADIX // 16, initix, 0)

  def initpb(j, carry):
    posb2[j // 8, pl.ds((j % 8) * 16, 16)] = iota16 * RADIX + t * DGT + j
    return carry

  lax.fori_loop(0, DGT * NT // 16, initpb, 0)

  for r in range(2):
    b = c * 2 + r
    for p in range(NPASS):
      shift = 10 * p
      if p % 2 == 0:
        dst_key, dst_idx = akey, aidx
        src_key, src_idx = bkey, bidx
      else:
        dst_key, dst_idx = bkey, bidx
        src_key, src_idx = akey, aidx
      if p == 0:
        pltpu.sync_copy(skey_hbm.at[pl.ds(b * NPAD + t * CHUNK, CHUNK)], keyv)

        def genidx(j, carry):
          idxv[pl.ds(j * 16, 16)] = t * CHUNK + j * 16 + iota16
          return carry

        lax.fori_loop(0, NVREG, genidx, 0)
      else:
        pltpu.sync_copy(src_key.at[pl.ds(t * CHUNK, CHUNK)], keyv)
        pltpu.sync_copy(src_idx.at[pl.ds(t * CHUNK, CHUNK)], idxv)
      for z in range(RADIX // 16):
        histv[pl.ds(z * 16, 16)] = zeros16

      def step_a(j, carry):
        k = keyv[pl.ds(j * 16, 16)]
        d = (k >> shift) & (RADIX - 1)
        ds, lanes = plsc.sort_key_val(d, iota16)
        tmpa[pl.ds(0, 16)] = ds
        prev = plsc.load_gather(tmpa, [idxm1])
        is_start = lane0 | (ds != prev)
        startpos = plsc.cummax(jnp.where(is_start, iota16, 0))
        runrank = iota16 - startpos
        cur = plsc.load_gather(histv, [ds])
        lrank_s = cur + runrank
        tmpb[pl.ds(0, 16)] = jnp.where(is_start, 1, 0)
        nxt = plsc.load_gather(tmpb, [idxp1])
        is_last = lane15 | (nxt == 1)
        plsc.store_scatter(histv, [ds], lrank_s + 1, mask=is_last)
        plsc.store_scatter(tmpc, [lanes], lrank_s)
        lrankv[pl.ds(j * 16, 16)] = tmpc[pl.ds(0, 16)]
        return carry

      lax.fori_loop(0, NVREG, step_a, 0)
      pend = [pltpu.async_copy(histv.at[pl.ds(g * 128, 128)],
                               grid_sh.at[pubix2.at[g]], sem)
              for g in range(RADIX // 128)]
      for dsc in pend:
        dsc.wait()
      plsc.subcore_barrier()
      pltpu.sync_copy(grid_sh.at[pl.ds(t * DGT * NT, DGT * NT)], scanv)

      def step_s(dl, run):
        cvec = scanv[pl.ds(dl * 16, 16)]
        inc = plsc.cumsum(cvec)
        tmpb[pl.ds(0, 16)] = inc
        tot = plsc.load_gather(tmpb, [splat15])
        scanv[pl.ds(dl * 16, 16)] = run + (inc - cvec)
        return run + tot

      blocktot = lax.fori_loop(0, DGT, step_s, zeros16)
      tmpa[pl.ds(0, 16)] = blocktot
      pltpu.sync_copy(tmpa, btot_sh.at[pl.ds(t * 16, 16)])
      pend = [pltpu.async_copy(scanv.at[pl.ds(g * 128, 128)],
                               base_sh.at[posb2.at[g]], sem)
              for g in range(DGT * NT // 128)]
      for dsc in pend:
        dsc.wait()
      plsc.subcore_barrier()
      pltpu.sync_copy(btot_sh, btotv)
      bts = plsc.load_gather(btotv, [iota16 * 16])
      boffs = plsc.cumsum(bts) - bts
      tmpc[pl.ds(0, 16)] = boffs
      pltpu.sync_copy(base_sh.at[pl.ds(t * RADIX, RADIX)], basev)

      def addoff(j, carry):
        off = plsc.load_gather(tmpc, [jnp.full((16,), j // (DGT // 16),
                                               jnp.int32)])
        basev[pl.ds(j * 16, 16)] = basev[pl.ds(j * 16, 16)] + off
        return carry

      lax.fori_loop(0, RADIX // 16, addoff, 0)

      def step_b(j, carry):
        k = keyv[pl.ds(j * 16, 16)]
        d = (k >> shift) & (RADIX - 1)
        lr = lrankv[pl.ds(j * 16, 16)]
        pos = plsc.load_gather(basev, [d]) + lr
        posv[j // 8, pl.ds((j % 8) * 16, 16)] = pos
        return carry

      lax.fori_loop(0, NVREG, step_b, 0)
      pend = []
      for q in range(NSCAT):
        if p != NPASS - 1:
          pend.append(pltpu.async_copy(keyv.at[pl.ds(q * 128, 128)],
                                       dst_key.at[posv.at[q]], sem))
        pend.append(pltpu.async_copy(idxv.at[pl.ds(q * 128, 128)],
                                     dst_idx.at[posv.at[q]], sem))
      for dsc in pend:
        dsc.wait()
      plsc.subcore_barrier()

    fin_idx = aidx if (NPASS - 1) % 2 == 0 else bidx
    pltpu.sync_copy(fin_idx.at[pl.ds(t * KPT, KPT)], gidxv)
    base_row = b * N
    base_key = b * NPAD

    def adj(j, carry):
      v = gidxv[pl.ds(j * 16, 16)] + base_row
      gidx2[j // 8, pl.ds((j % 8) * 16, 16)] = v
      return carry

    lax.fori_loop(0, KPT // 16, adj, 0)
    pend = [
        pltpu.async_copy(h_hbm.at[gidx2.at[g]],
                         rowsv.at[pl.ds(g * 128, 128)], sem)
        for g in range(KPT // 128)
    ]
    for dsc in pend:
      dsc.wait()

    def adjk(j, carry):
      v = gidxv[pl.ds(j * 16, 16)] + base_key
      gidx2[j // 8, pl.ds((j % 8) * 16, 16)] = v
      return carry

    lax.fori_loop(0, KPT // 16, adjk, 0)
    pend = [
        pltpu.async_copy(skey_hbm.at[gidx2.at[g]],
                         gkeyv.at[pl.ds(g * 128, 128)], sem)
        for g in range(KPT // 128)
    ]
    for dsc in pend:
      dsc.wait()

    def scale(j, carry):
      kv = plsc.load_gather(gkeyv, [jnp.full((16,), j, jnp.int32)])
      sc = plsc.bitcast(jnp.full((16,), SENT, jnp.int32) - kv, jnp.float32)
      for q in range(8):
        rowsv[j, pl.ds(q * 16, 16)] = rowsv[j, pl.ds(q * 16, 16)] * sc
      return carry

    lax.fori_loop(0, KPT, scale, 0)
    pltpu.sync_copy(rowsv, out_hbm.at[pl.ds(b * K + t * KPT, KPT)])


@functools.partial(
    pl.kernel,
    out_type=jax.ShapeDtypeStruct((B * K, D), jnp.float32),
    mesh=plsc.VectorSubcoreMesh(core_axis_name="c", subcore_axis_name="s",
                                num_cores=NC),
    compiler_params=pltpu.CompilerParams(needs_layout_passes=False),
    scratch_types=[
        pltpu.VMEM_SHARED((NPAD,), jnp.int32),
        pltpu.VMEM_SHARED((NPAD,), jnp.int32),
        pltpu.VMEM_SHARED((NPAD,), jnp.int32),
        pltpu.VMEM_SHARED((NPAD,), jnp.int32),
        pltpu.VMEM_SHARED((NT * RADIX,), jnp.int32),
        pltpu.VMEM_SHARED((NT * RADIX,), jnp.int32),
        pltpu.VMEM_SHARED((NT * 16,), jnp.int32),
        pltpu.VMEM((CHUNK,), jnp.int32),
        pltpu.VMEM((CHUNK,), jnp.int32),
        pltpu.VMEM((CHUNK,), jnp.int32),
        pltpu.VMEM((NSCAT, 128), jnp.int32),
        pltpu.VMEM((RADIX,), jnp.int32),
        pltpu.VMEM((RADIX,), jnp.int32),
        pltpu.VMEM((DGT * NT,), jnp.int32),
        pltpu.VMEM((RADIX // 128, 128), jnp.int32),
        pltpu.VMEM((DGT * NT // 128, 128), jnp.int32),
        pltpu.VMEM((NT * 16,), jnp.int32),
        pltpu.VMEM((16,), jnp.int32),
        pltpu.VMEM((16,), jnp.int32),
        pltpu.VMEM((16,), jnp.int32),
        pltpu.VMEM((KPT,), jnp.int32),
        pltpu.VMEM((KPT,), jnp.int32),
        pltpu.VMEM((KPT // 128, 128), jnp.int32),
        pltpu.VMEM((KPT, D), jnp.float32),
        pltpu.SemaphoreType.DMA,
    ],
)
def _sc_topk(skey_hbm, h_hbm, out_hbm, *rest):
  _sc_body(skey_hbm, h_hbm, out_hbm, *rest)


def kernel(h, section_feature):
  skey = _scores(h, section_feature)
  skey = jnp.concatenate(
      [skey, jnp.full((B, NPAD - N), SENT, jnp.int32)], axis=1)
  out = _sc_topk(skey.reshape(B * NPAD), h.reshape(B * N, D))
  return out.reshape(B, K, D)


# speedup vs baseline: 2.9002x; 1.0402x over previous
import functools

import jax
import jax.numpy as jnp
from jax import lax
from jax.experimental import pallas as pl
from jax.experimental.pallas import tpu as pltpu
from jax.experimental.pallas import tpu_sc as plsc

B = 4
N = 50000
D = 128
K = 8192

NT = 16
NC = 2
NPAD = 51200
CHUNK = NPAD // NT
NVREG = CHUNK // 16
RADIX = 1024
NPASS = 3
DGT = RADIX // NT
NSCAT = CHUNK // 128
KPT = K // NT
SENT = 0x3FFFFFFF

BLKN = 5000


def _score_body(h_ref, sf_ref, out_ref):
  w = lax.dot_general(
      sf_ref[0], h_ref[0],
      dimension_numbers=(((1,), (1,)), ((), ())),
      preferred_element_type=jnp.float32,
  )
  score = pl.reciprocal(1.0 + jnp.exp(-w), approx=False)
  skey = SENT - lax.bitcast_convert_type(score, jnp.int32)
  out_ref[0] = skey


def _scores(h, section_feature):
  grid = (B, N // BLKN)
  out = pl.pallas_call(
      _score_body,
      grid=grid,
      in_specs=[
          pl.BlockSpec((1, BLKN, D), lambda b, j: (b, j, 0)),
          pl.BlockSpec((1, 1, D), lambda b, j: (b, 0, 0)),
      ],
      out_specs=pl.BlockSpec((1, 1, BLKN), lambda b, j: (b * (N // BLKN) + j, 0, 0)),
      out_shape=jax.ShapeDtypeStruct((B * (N // BLKN), 1, BLKN), jnp.int32),
  )(h, section_feature)
  return out.reshape(B, N)


def _sc_body(skey_hbm, h_hbm, out_hbm,
             akey, aidx, bkey, bidx, grid_sh, base_sh, btot_sh,
             keyv, idxv, lrankv, posv, histv, basev, scanv,
             pubix2, posb2, btotv,
             tmpa, tmpb, tmpc, gkeyv, gidxv, gidx2, rowsv, sem):
  c = lax.axis_index("c")
  t = lax.axis_index("s")
  iota16 = lax.iota(jnp.int32, 16)
  lane0 = iota16 == 0
  lane15 = iota16 == 15
  idxm1 = jnp.maximum(iota16 - 1, 0)
  idxp1 = jnp.minimum(iota16 + 1, 15)
  splat15 = jnp.full((16,), 15, jnp.int32)
  splat_t = jnp.full((16,), t, jnp.int32)
  iota_r = iota16 * RADIX
  zeros16 = jnp.zeros((16,), jnp.int32)

  def initix(j, carry):
    pubix2[j // 8, pl.ds((j % 8) * 16, 16)] = (j * 16 + iota16) * NT + t
    return carry

  lax.fori_loop(0, RADIX // 16, initix, 0)

  def initpb(j, carry):
    posb2[j // 8, pl.ds((j % 8) * 16, 16)] = iota16 * RADIX + t * DGT + j
    return carry

  lax.fori_loop(0, DGT * NT // 16, initpb, 0)

  for r in range(2):
    b = c * 2 + r
    for p in range(NPASS):
      shift = 10 * p
      if p % 2 == 0:
        dst_key, dst_idx = akey, aidx
        src_key, src_idx = bkey, bidx
      else:
        dst_key, dst_idx = bkey, bidx
        src_key, src_idx = akey, aidx
      if p == 0:
        pltpu.sync_copy(skey_hbm.at[pl.ds(b * NPAD + t * CHUNK, CHUNK)], keyv)

        def genidx(j, carry):
          idxv[pl.ds(j * 16, 16)] = t * CHUNK + j * 16 + iota16
          return carry

        lax.fori_loop(0, NVREG, genidx, 0)
      else:
        pltpu.sync_copy(src_key.at[pl.ds(t * CHUNK, CHUNK)], keyv)
        pltpu.sync_copy(src_idx.at[pl.ds(t * CHUNK, CHUNK)], idxv)
      for z in range(RADIX // 16):
        histv[pl.ds(z * 16, 16)] = zeros16

      def step_a(jj, carry):
        for u in range(4):
          j = jj * 4 + u
          k = keyv[pl.ds(j * 16, 16)]
          d = (k >> shift) & (RADIX - 1)
          ds, lanes = plsc.sort_key_val(d, iota16)
          inv = plsc.sort_key_val(lanes, iota16)[1]
          prev = jnp.take_along_axis(ds, idxm1, axis=0, mode=lax.GatherScatterMode.PROMISE_IN_BOUNDS)
          is_start = lane0 | (ds != prev)
          startpos = plsc.cummax(jnp.where(is_start, iota16, 0))
          runrank = iota16 - startpos
          cur = plsc.load_gather(histv, [ds])
          lrank_s = cur + runrank
          st = jnp.where(is_start, 1, 0)
          nxt = jnp.take_along_axis(st, idxp1, axis=0, mode=lax.GatherScatterMode.PROMISE_IN_BOUNDS)
          is_last = lane15 | (nxt == 1)
          plsc.store_scatter(histv, [ds], lrank_s + 1, mask=is_last)
          lrankv[pl.ds(j * 16, 16)] = jnp.take_along_axis(
              lrank_s, inv, axis=0,
              mode=lax.GatherScatterMode.PROMISE_IN_BOUNDS)
        return carry

      lax.fori_loop(0, NVREG // 4, step_a, 0)
      pend = [pltpu.async_copy(histv.at[pl.ds(g * 128, 128)],
                               grid_sh.at[pubix2.at[g]], sem)
              for g in range(RADIX // 128)]
      for dsc in pend:
        dsc.wait()
      plsc.subcore_barrier()
      pltpu.sync_copy(grid_sh.at[pl.ds(t * DGT * NT, DGT * NT)], scanv)

      def step_s(dl, run):
        cvec = scanv[pl.ds(dl * 16, 16)]
        inc = plsc.cumsum(cvec)
        tmpb[pl.ds(0, 16)] = inc
        tot = plsc.load_gather(tmpb, [splat15])
        scanv[pl.ds(dl * 16, 16)] = run + (inc - cvec)
        return run + tot

      blocktot = lax.fori_loop(0, DGT, step_s, zeros16)
      tmpa[pl.ds(0, 16)] = blocktot
      pltpu.sync_copy(tmpa, btot_sh.at[pl.ds(t * 16, 16)])
      pend = [pltpu.async_copy(scanv.at[pl.ds(g * 128, 128)],
                               base_sh.at[posb2.at[g]], sem)
              for g in range(DGT * NT // 128)]
      for dsc in pend:
        dsc.wait()
      plsc.subcore_barrier()
      pltpu.sync_copy(btot_sh, btotv)
      bts = plsc.load_gather(btotv, [iota16 * 16])
      boffs = plsc.cumsum(bts) - bts
      tmpc[pl.ds(0, 16)] = boffs
      pltpu.sync_copy(base_sh.at[pl.ds(t * RADIX, RADIX)], basev)

      def addoff(j, carry):
        off = plsc.load_gather(tmpc, [jnp.full((16,), j // (DGT // 16),
                                               jnp.int32)])
        basev[pl.ds(j * 16, 16)] = basev[pl.ds(j * 16, 16)] + off
        return carry

      lax.fori_loop(0, RADIX // 16, addoff, 0)

      def step_b(jj, carry):
        for u in range(4):
          j = jj * 4 + u
          k = keyv[pl.ds(j * 16, 16)]
          d = (k >> shift) & (RADIX - 1)
          lr = lrankv[pl.ds(j * 16, 16)]
          pos = plsc.load_gather(basev, [d]) + lr
          posv[j // 8, pl.ds((j % 8) * 16, 16)] = pos
        return carry

      lax.fori_loop(0, NVREG // 4, step_b, 0)
      pend = []
      for q in range(NSCAT):
        if p != NPASS - 1:
          pend.append(pltpu.async_copy(keyv.at[pl.ds(q * 128, 128)],
                                       dst_key.at[posv.at[q]], sem))
        pend.append(pltpu.async_copy(idxv.at[pl.ds(q * 128, 128)],
                                     dst_idx.at[posv.at[q]], sem))
      for dsc in pend:
        dsc.wait()
      plsc.subcore_barrier()

    fin_idx = aidx if (NPASS - 1) % 2 == 0 else bidx
    pltpu.sync_copy(fin_idx.at[pl.ds(t * KPT, KPT)], gidxv)
    base_row = b * N
    base_key = b * NPAD

    def adj(j, carry):
      v = gidxv[pl.ds(j * 16, 16)] + base_row
      gidx2[j // 8, pl.ds((j % 8) * 16, 16)] = v
      return carry

    lax.fori_loop(0, KPT // 16, adj, 0)
    pend = [
        pltpu.async_copy(h_hbm.at[gidx2.at[g]],
                         rowsv.at[pl.ds(g * 128, 128)], sem)
        for g in range(KPT // 128)
    ]
    for dsc in pend:
      dsc.wait()

    def adjk(j, carry):
      v = gidxv[pl.ds(j * 16, 16)] + base_key
      gidx2[j // 8, pl.ds((j % 8) * 16, 16)] = v
      return carry

    lax.fori_loop(0, KPT // 16, adjk, 0)
    pend = [
        pltpu.async_copy(skey_hbm.at[gidx2.at[g]],
                         gkeyv.at[pl.ds(g * 128, 128)], sem)
        for g in range(KPT // 128)
    ]
    for dsc in pend:
      dsc.wait()

    def scale(jj, carry):
      for u in range(2):
        j = jj * 2 + u
        kv = plsc.load_gather(gkeyv, [jnp.full((16,), j, jnp.int32)])
        sc = plsc.bitcast(jnp.full((16,), SENT, jnp.int32) - kv, jnp.float32)
        for q in range(8):
          rowsv[j, pl.ds(q * 16, 16)] = rowsv[j, pl.ds(q * 16, 16)] * sc
      return carry

    lax.fori_loop(0, KPT // 2, scale, 0)
    pltpu.sync_copy(rowsv, out_hbm.at[pl.ds(b * K + t * KPT, KPT)])


@functools.partial(
    pl.kernel,
    out_type=jax.ShapeDtypeStruct((B * K, D), jnp.float32),
    mesh=plsc.VectorSubcoreMesh(core_axis_name="c", subcore_axis_name="s",
                                num_cores=NC),
    compiler_params=pltpu.CompilerParams(needs_layout_passes=False),
    scratch_types=[
        pltpu.VMEM_SHARED((NPAD,), jnp.int32),
        pltpu.VMEM_SHARED((NPAD,), jnp.int32),
        pltpu.VMEM_SHARED((NPAD,), jnp.int32),
        pltpu.VMEM_SHARED((NPAD,), jnp.int32),
        pltpu.VMEM_SHARED((NT * RADIX,), jnp.int32),
        pltpu.VMEM_SHARED((NT * RADIX,), jnp.int32),
        pltpu.VMEM_SHARED((NT * 16,), jnp.int32),
        pltpu.VMEM((CHUNK,), jnp.int32),
        pltpu.VMEM((CHUNK,), jnp.int32),
        pltpu.VMEM((CHUNK,), jnp.int32),
        pltpu.VMEM((NSCAT, 128), jnp.int32),
        pltpu.VMEM((RADIX,), jnp.int32),
        pltpu.VMEM((RADIX,), jnp.int32),
        pltpu.VMEM((DGT * NT,), jnp.int32),
        pltpu.VMEM((RADIX // 128, 128), jnp.int32),
        pltpu.VMEM((DGT * NT // 128, 128), jnp.int32),
        pltpu.VMEM((NT * 16,), jnp.int32),
        pltpu.VMEM((16,), jnp.int32),
        pltpu.VMEM((16,), jnp.int32),
        pltpu.VMEM((16,), jnp.int32),
        pltpu.VMEM((KPT,), jnp.int32),
        pltpu.VMEM((KPT,), jnp.int32),
        pltpu.VMEM((KPT // 128, 128), jnp.int32),
        pltpu.VMEM((KPT, D), jnp.float32),
        pltpu.SemaphoreType.DMA,
    ],
)
def _sc_topk(skey_hbm, h_hbm, out_hbm, *rest):
  _sc_body(skey_hbm, h_hbm, out_hbm, *rest)


def kernel(h, section_feature):
  skey = _scores(h, section_feature)
  skey = jnp.concatenate(
      [skey, jnp.full((B, NPAD - N), SENT, jnp.int32)], axis=1)
  out = _sc_topk(skey.reshape(B * NPAD), h.reshape(B * N, D))
  return out.reshape(B, K, D)


# speedup vs baseline: 2.9814x; 1.0280x over previous
import functools

import jax
import jax.numpy as jnp
from jax import lax
from jax.experimental import pallas as pl
from jax.experimental.pallas import tpu as pltpu
from jax.experimental.pallas import tpu_sc as plsc

B = 4
N = 50000
D = 128
K = 8192

NT = 16
NC = 2
NPAD = 51200
CHUNK = NPAD // NT
NVREG = CHUNK // 16
RADIX = 1024
NPASS = 3
DGT = RADIX // NT
NSCAT = CHUNK // 128
KPT = K // NT
SENT = 0x3FFFFFFF

BLKN = 5000


def _score_body(h_ref, sf_ref, out_ref):
  w = lax.dot_general(
      sf_ref[0], h_ref[0],
      dimension_numbers=(((1,), (1,)), ((), ())),
      preferred_element_type=jnp.float32,
  )
  score = pl.reciprocal(1.0 + jnp.exp(-w), approx=False)
  skey = SENT - lax.bitcast_convert_type(score, jnp.int32)
  out_ref[0] = skey


def _scores(h, section_feature, boff):
  grid = (2, N // BLKN)
  out = pl.pallas_call(
      _score_body,
      grid=grid,
      in_specs=[
          pl.BlockSpec((1, BLKN, D), lambda b, j: (b + boff, j, 0)),
          pl.BlockSpec((1, 1, D), lambda b, j: (b + boff, 0, 0)),
      ],
      out_specs=pl.BlockSpec((1, 1, BLKN),
                             lambda b, j: (b * (N // BLKN) + j, 0, 0)),
      out_shape=jax.ShapeDtypeStruct((2 * (N // BLKN), 1, BLKN), jnp.int32),
  )(h, section_feature)
  return out.reshape(2, N)


def _sc_body(boff, skey_hbm, h_hbm, out_hbm,
             akey, aidx, bkey, bidx, grid_sh, base_sh, btot_sh,
             keyv, idxv, lrankv, posv, histv, basev, scanv,
             pubix2, posb2, btotv,
             tmpa, tmpb, tmpc, gkeyv, gidxv, gidx2, rowsv, sem):
  c = lax.axis_index("c")
  t = lax.axis_index("s")
  iota16 = lax.iota(jnp.int32, 16)
  lane0 = iota16 == 0
  lane15 = iota16 == 15
  idxm1 = jnp.maximum(iota16 - 1, 0)
  idxp1 = jnp.minimum(iota16 + 1, 15)
  splat15 = jnp.full((16,), 15, jnp.int32)
  splat_t = jnp.full((16,), t, jnp.int32)
  iota_r = iota16 * RADIX
  zeros16 = jnp.zeros((16,), jnp.int32)

  def initix(j, carry):
    pubix2[j // 8, pl.ds((j % 8) * 16, 16)] = (j * 16 + iota16) * NT + t
    return carry

  lax.fori_loop(0, RADIX // 16, initix, 0)

  def initpb(j, carry):
    posb2[j // 8, pl.ds((j % 8) * 16, 16)] = iota16 * RADIX + t * DGT + j
    return carry

  lax.fori_loop(0, DGT * NT // 16, initpb, 0)

  if True:
    b = c
    for p in range(NPASS):
      shift = 10 * p
      if p % 2 == 0:
        dst_key, dst_idx = akey, aidx
        src_key, src_idx = bkey, bidx
      else:
        dst_key, dst_idx = bkey, bidx
        src_key, src_idx = akey, aidx
      if p == 0:
        pltpu.sync_copy(skey_hbm.at[pl.ds(b * NPAD + t * CHUNK, CHUNK)], keyv)

        def genidx(j, carry):
          idxv[pl.ds(j * 16, 16)] = t * CHUNK + j * 16 + iota16
          return carry

        lax.fori_loop(0, NVREG, genidx, 0)
      else:
        pltpu.sync_copy(src_key.at[pl.ds(t * CHUNK, CHUNK)], keyv)
        pltpu.sync_copy(src_idx.at[pl.ds(t * CHUNK, CHUNK)], idxv)
      for z in range(RADIX // 16):
        histv[pl.ds(z * 16, 16)] = zeros16

      def step_a(jj, carry):
        for u in range(4):
          j = jj * 4 + u
          k = keyv[pl.ds(j * 16, 16)]
          d = (k >> shift) & (RADIX - 1)
          ds, lanes = plsc.sort_key_val(d, iota16)
          inv = plsc.sort_key_val(lanes, iota16)[1]
          prev = jnp.take_along_axis(ds, idxm1, axis=0, mode=lax.GatherScatterMode.PROMISE_IN_BOUNDS)
          is_start = lane0 | (ds != prev)
          startpos = plsc.cummax(jnp.where(is_start, iota16, 0))
          runrank = iota16 - startpos
          cur = plsc.load_gather(histv, [ds])
          lrank_s = cur + runrank
          st = jnp.where(is_start, 1, 0)
          nxt = jnp.take_along_axis(st, idxp1, axis=0, mode=lax.GatherScatterMode.PROMISE_IN_BOUNDS)
          is_last = lane15 | (nxt == 1)
          plsc.store_scatter(histv, [ds], lrank_s + 1, mask=is_last)
          lrankv[pl.ds(j * 16, 16)] = jnp.take_along_axis(
              lrank_s, inv, axis=0,
              mode=lax.GatherScatterMode.PROMISE_IN_BOUNDS)
        return carry

      lax.fori_loop(0, NVREG // 4, step_a, 0)
      pend = [pltpu.async_copy(histv.at[pl.ds(g * 128, 128)],
                               grid_sh.at[pubix2.at[g]], sem)
              for g in range(RADIX // 128)]
      for dsc in pend:
        dsc.wait()
      plsc.subcore_barrier()
      pltpu.sync_copy(grid_sh.at[pl.ds(t * DGT * NT, DGT * NT)], scanv)

      def step_s(dl, run):
        cvec = scanv[pl.ds(dl * 16, 16)]
        inc = plsc.cumsum(cvec)
        tmpb[pl.ds(0, 16)] = inc
        tot = plsc.load_gather(tmpb, [splat15])
        scanv[pl.ds(dl * 16, 16)] = run + (inc - cvec)
        return run + tot

      blocktot = lax.fori_loop(0, DGT, step_s, zeros16)
      tmpa[pl.ds(0, 16)] = blocktot
      pltpu.sync_copy(tmpa, btot_sh.at[pl.ds(t * 16, 16)])
      pend = [pltpu.async_copy(scanv.at[pl.ds(g * 128, 128)],
                               base_sh.at[posb2.at[g]], sem)
              for g in range(DGT * NT // 128)]
      for dsc in pend:
        dsc.wait()
      plsc.subcore_barrier()
      pltpu.sync_copy(btot_sh, btotv)
      bts = plsc.load_gather(btotv, [iota16 * 16])
      boffs = plsc.cumsum(bts) - bts
      tmpc[pl.ds(0, 16)] = boffs
      pltpu.sync_copy(base_sh.at[pl.ds(t * RADIX, RADIX)], basev)

      def addoff(j, carry):
        off = plsc.load_gather(tmpc, [jnp.full((16,), j // (DGT // 16),
                                               jnp.int32)])
        basev[pl.ds(j * 16, 16)] = basev[pl.ds(j * 16, 16)] + off
        return carry

      lax.fori_loop(0, RADIX // 16, addoff, 0)

      def step_b(jj, carry):
        for u in range(4):
          j = jj * 4 + u
          k = keyv[pl.ds(j * 16, 16)]
          d = (k >> shift) & (RADIX - 1)
          lr = lrankv[pl.ds(j * 16, 16)]
          pos = plsc.load_gather(basev, [d]) + lr
          posv[j // 8, pl.ds((j % 8) * 16, 16)] = pos
        return carry

      lax.fori_loop(0, NVREG // 4, step_b, 0)
      pend = []
      for q in range(NSCAT):
        if p != NPASS - 1:
          pend.append(pltpu.async_copy(keyv.at[pl.ds(q * 128, 128)],
                                       dst_key.at[posv.at[q]], sem))
        pend.append(pltpu.async_copy(idxv.at[pl.ds(q * 128, 128)],
                                     dst_idx.at[posv.at[q]], sem))
      for dsc in pend:
        dsc.wait()
      plsc.subcore_barrier()

    fin_idx = aidx if (NPASS - 1) % 2 == 0 else bidx
    pltpu.sync_copy(fin_idx.at[pl.ds(t * KPT, KPT)], gidxv)
    base_row = (boff + b) * N
    base_key = b * NPAD

    def adj(j, carry):
      v = gidxv[pl.ds(j * 16, 16)] + base_row
      gidx2[j // 8, pl.ds((j % 8) * 16, 16)] = v
      return carry

    lax.fori_loop(0, KPT // 16, adj, 0)
    pend = [
        pltpu.async_copy(h_hbm.at[gidx2.at[g]],
                         rowsv.at[pl.ds(g * 128, 128)], sem)
        for g in range(KPT // 128)
    ]
    for dsc in pend:
      dsc.wait()

    def adjk(j, carry):
      v = gidxv[pl.ds(j * 16, 16)] + base_key
      gidx2[j // 8, pl.ds((j % 8) * 16, 16)] = v
      return carry

    lax.fori_loop(0, KPT // 16, adjk, 0)
    pend = [
        pltpu.async_copy(skey_hbm.at[gidx2.at[g]],
                         gkeyv.at[pl.ds(g * 128, 128)], sem)
        for g in range(KPT // 128)
    ]
    for dsc in pend:
      dsc.wait()

    def scale(jj, carry):
      for u in range(2):
        j = jj * 2 + u
        kv = plsc.load_gather(gkeyv, [jnp.full((16,), j, jnp.int32)])
        sc = plsc.bitcast(jnp.full((16,), SENT, jnp.int32) - kv, jnp.float32)
        for q in range(8):
          rowsv[j, pl.ds(q * 16, 16)] = rowsv[j, pl.ds(q * 16, 16)] * sc
      return carry

    lax.fori_loop(0, KPT // 2, scale, 0)
    pltpu.sync_copy(rowsv, out_hbm.at[pl.ds(b * K + t * KPT, KPT)])


_SC_SCRATCH = [
        pltpu.VMEM_SHARED((NPAD,), jnp.int32),
        pltpu.VMEM_SHARED((NPAD,), jnp.int32),
        pltpu.VMEM_SHARED((NPAD,), jnp.int32),
        pltpu.VMEM_SHARED((NPAD,), jnp.int32),
        pltpu.VMEM_SHARED((NT * RADIX,), jnp.int32),
        pltpu.VMEM_SHARED((NT * RADIX,), jnp.int32),
        pltpu.VMEM_SHARED((NT * 16,), jnp.int32),
        pltpu.VMEM((CHUNK,), jnp.int32),
        pltpu.VMEM((CHUNK,), jnp.int32),
        pltpu.VMEM((CHUNK,), jnp.int32),
        pltpu.VMEM((NSCAT, 128), jnp.int32),
        pltpu.VMEM((RADIX,), jnp.int32),
        pltpu.VMEM((RADIX,), jnp.int32),
        pltpu.VMEM((DGT * NT,), jnp.int32),
        pltpu.VMEM((RADIX // 128, 128), jnp.int32),
        pltpu.VMEM((DGT * NT // 128, 128), jnp.int32),
        pltpu.VMEM((NT * 16,), jnp.int32),
        pltpu.VMEM((16,), jnp.int32),
        pltpu.VMEM((16,), jnp.int32),
        pltpu.VMEM((16,), jnp.int32),
        pltpu.VMEM((KPT,), jnp.int32),
        pltpu.VMEM((KPT,), jnp.int32),
        pltpu.VMEM((KPT // 128, 128), jnp.int32),
        pltpu.VMEM((KPT, D), jnp.float32),
        pltpu.SemaphoreType.DMA,
]


def _make_sc_topk(boff):
  @functools.partial(
      pl.kernel,
      out_type=jax.ShapeDtypeStruct((2 * K, D), jnp.float32),
      mesh=plsc.VectorSubcoreMesh(core_axis_name="c", subcore_axis_name="s",
                                  num_cores=NC),
      compiler_params=pltpu.CompilerParams(needs_layout_passes=False),
      scratch_types=_SC_SCRATCH,
  )
  def _sc_topk(skey_hbm, h_hbm, out_hbm, *rest):
    _sc_body(boff, skey_hbm, h_hbm, out_hbm, *rest)

  return _sc_topk


_sc_topk01 = _make_sc_topk(0)
_sc_topk23 = _make_sc_topk(2)


def kernel(h, section_feature):
  h2d = h.reshape(B * N, D)
  pad = jnp.full((2, NPAD - N), SENT, jnp.int32)
  s01 = jnp.concatenate([_scores(h, section_feature, 0), pad], axis=1)
  s23 = jnp.concatenate([_scores(h, section_feature, 2), pad], axis=1)
  o01 = _sc_topk01(s01.reshape(2 * NPAD), h2d)
  o23 = _sc_topk23(s23.reshape(2 * NPAD), h2d)
  return jnp.concatenate([o01, o23], axis=0).reshape(B, K, D)


# speedup vs baseline: 3.0878x; 1.0357x over previous
import functools

import jax
import jax.numpy as jnp
from jax import lax
from jax.experimental import pallas as pl
from jax.experimental.pallas import tpu as pltpu
from jax.experimental.pallas import tpu_sc as plsc

B = 4
N = 50000
D = 128
K = 8192

NT = 16
NC = 2
NPAD = 51200
CHUNK = NPAD // NT
NVREG = CHUNK // 16
RADIX = 1024
NPASS = 3
DGT = RADIX // NT
NSCAT = CHUNK // 128
KPT = K // NT
SENT = 0x3FFFFFFF

BLKN = 5000


def _score_body(h_ref, sf_ref, out_ref):
  w = lax.dot_general(
      sf_ref[0], h_ref[0],
      dimension_numbers=(((1,), (1,)), ((), ())),
      preferred_element_type=jnp.float32,
  )
  score = pl.reciprocal(1.0 + jnp.exp(-w), approx=False)
  skey = SENT - lax.bitcast_convert_type(score, jnp.int32)
  out_ref[0] = skey


def _scores(h, section_feature, boff):
  grid = (2, N // BLKN)
  out = pl.pallas_call(
      _score_body,
      grid=grid,
      in_specs=[
          pl.BlockSpec((1, BLKN, D), lambda b, j: (b + boff, j, 0)),
          pl.BlockSpec((1, 1, D), lambda b, j: (b + boff, 0, 0)),
      ],
      out_specs=pl.BlockSpec((1, 1, BLKN),
                             lambda b, j: (b * (N // BLKN) + j, 0, 0)),
      out_shape=jax.ShapeDtypeStruct((2 * (N // BLKN), 1, BLKN), jnp.int32),
  )(h, section_feature)
  return out.reshape(2, N)


def _sc_body(boff, skey_hbm, h_hbm, out_hbm,
             akey, aidx, bkey, bidx, grid_sh, base_sh, btot_sh,
             keyv, idxv, lrankv, posv, histv, basev, scanv,
             pubix2, posb2, btotv,
             tmpa, tmpb, tmpc, gkeyv, gidxv, gidx2, rowsv, sem):
  c = lax.axis_index("c")
  t = lax.axis_index("s")
  iota16 = lax.iota(jnp.int32, 16)
  lane0 = iota16 == 0
  lane15 = iota16 == 15
  idxm1 = jnp.maximum(iota16 - 1, 0)
  idxp1 = jnp.minimum(iota16 + 1, 15)
  splat15 = jnp.full((16,), 15, jnp.int32)
  splat_t = jnp.full((16,), t, jnp.int32)
  iota_r = iota16 * RADIX
  zeros16 = jnp.zeros((16,), jnp.int32)

  def initix(j, carry):
    pubix2[j // 8, pl.ds((j % 8) * 16, 16)] = (j * 16 + iota16) * NT + t
    return carry

  lax.fori_loop(0, RADIX // 16, initix, 0)

  def initpb(j, carry):
    posb2[j // 8, pl.ds((j % 8) * 16, 16)] = iota16 * RADIX + t * DGT + j
    return carry

  lax.fori_loop(0, DGT * NT // 16, initpb, 0)

  if True:
    b = c
    for p in range(NPASS):
      shift = 10 * p
      if p % 2 == 0:
        dst_key, dst_idx = akey, aidx
        src_key, src_idx = bkey, bidx
      else:
        dst_key, dst_idx = bkey, bidx
        src_key, src_idx = akey, aidx
      if p == 0:
        pltpu.sync_copy(skey_hbm.at[pl.ds(b * NPAD + t * CHUNK, CHUNK)], keyv)

        def genidx(j, carry):
          idxv[pl.ds(j * 16, 16)] = t * CHUNK + j * 16 + iota16
          return carry

        lax.fori_loop(0, NVREG, genidx, 0)
      else:
        pltpu.sync_copy(src_key.at[pl.ds(t * CHUNK, CHUNK)], keyv)
        pltpu.sync_copy(src_idx.at[pl.ds(t * CHUNK, CHUNK)], idxv)
      for z in range(RADIX // 16):
        histv[pl.ds(z * 16, 16)] = zeros16

      def step_a(jj, carry):
        for u in range(4):
          j = jj * 4 + u
          k = keyv[pl.ds(j * 16, 16)]
          d = (k >> shift) & (RADIX - 1)
          ds, lanes = plsc.sort_key_val(d, iota16)
          inv = plsc.sort_key_val(lanes, iota16)[1]
          prev = jnp.take_along_axis(ds, idxm1, axis=0, mode=lax.GatherScatterMode.PROMISE_IN_BOUNDS)
          is_start = lane0 | (ds != prev)
          startpos = plsc.cummax(jnp.where(is_start, iota16, 0))
          runrank = iota16 - startpos
          cur = plsc.load_gather(histv, [ds])
          lrank_s = cur + runrank
          st = jnp.where(is_start, 1, 0)
          nxt = jnp.take_along_axis(st, idxp1, axis=0, mode=lax.GatherScatterMode.PROMISE_IN_BOUNDS)
          is_last = lane15 | (nxt == 1)
          plsc.store_scatter(histv, [ds], lrank_s + 1, mask=is_last)
          lrankv[pl.ds(j * 16, 16)] = jnp.take_along_axis(
              lrank_s, inv, axis=0,
              mode=lax.GatherScatterMode.PROMISE_IN_BOUNDS)
        return carry

      lax.fori_loop(0, NVREG // 4, step_a, 0)
      pend = [pltpu.async_copy(histv.at[pl.ds(g * 128, 128)],
                               grid_sh.at[pubix2.at[g]], sem)
              for g in range(RADIX // 128)]
      for dsc in pend:
        dsc.wait()
      plsc.subcore_barrier()
      pltpu.sync_copy(grid_sh.at[pl.ds(t * DGT * NT, DGT * NT)], scanv)

      def step_s(dl, run):
        cvec = scanv[pl.ds(dl * 16, 16)]
        inc = plsc.cumsum(cvec)
        tmpb[pl.ds(0, 16)] = inc
        tot = plsc.load_gather(tmpb, [splat15])
        scanv[pl.ds(dl * 16, 16)] = run + (inc - cvec)
        return run + tot

      blocktot = lax.fori_loop(0, DGT, step_s, zeros16)
      tmpa[pl.ds(0, 16)] = blocktot
      pltpu.sync_copy(tmpa, btot_sh.at[pl.ds(t * 16, 16)])
      pend = [pltpu.async_copy(scanv.at[pl.ds(g * 128, 128)],
                               base_sh.at[posb2.at[g]], sem)
              for g in range(DGT * NT // 128)]
      for dsc in pend:
        dsc.wait()
      plsc.subcore_barrier()
      pltpu.sync_copy(btot_sh, btotv)
      bts = plsc.load_gather(btotv, [iota16 * 16])
      boffs = plsc.cumsum(bts) - bts
      tmpc[pl.ds(0, 16)] = boffs
      pltpu.sync_copy(base_sh.at[pl.ds(t * RADIX, RADIX)], basev)

      def addoff(j, carry):
        off = plsc.load_gather(tmpc, [jnp.full((16,), j // (DGT // 16),
                                               jnp.int32)])
        basev[pl.ds(j * 16, 16)] = basev[pl.ds(j * 16, 16)] + off
        return carry

      lax.fori_loop(0, RADIX // 16, addoff, 0)

      def step_b(jj, carry):
        for u in range(4):
          j = jj * 4 + u
          k = keyv[pl.ds(j * 16, 16)]
          d = (k >> shift) & (RADIX - 1)
          lr = lrankv[pl.ds(j * 16, 16)]
          pos = plsc.load_gather(basev, [d]) + lr
          posv[j // 8, pl.ds((j % 8) * 16, 16)] = pos
        return carry

      lax.fori_loop(0, NVREG // 4, step_b, 0)
      pend = []
      for q in range(NSCAT):
        if p != NPASS - 1:
          pend.append(pltpu.async_copy(keyv.at[pl.ds(q * 128, 128)],
                                       dst_key.at[posv.at[q]], sem))
        pend.append(pltpu.async_copy(idxv.at[pl.ds(q * 128, 128)],
                                     dst_idx.at[posv.at[q]], sem))
      for dsc in pend:
        dsc.wait()
      plsc.subcore_barrier()

    fin_idx = aidx if (NPASS - 1) % 2 == 0 else bidx
    pltpu.sync_copy(fin_idx.at[pl.ds(t * KPT, KPT)], gidxv)
    base_row = (boff + b) * N
    base_key = b * NPAD

    def adj(j, carry):
      v = gidxv[pl.ds(j * 16, 16)] + base_row
      gidx2[j // 8, pl.ds((j % 8) * 16, 16)] = v
      return carry

    lax.fori_loop(0, KPT // 16, adj, 0)
    pend = [
        pltpu.async_copy(h_hbm.at[gidx2.at[g]],
                         rowsv.at[pl.ds(g * 128, 128)], sem)
        for g in range(KPT // 128)
    ]
    for dsc in pend:
      dsc.wait()

    def adjk(j, carry):
      v = gidxv[pl.ds(j * 16, 16)] + base_key
      gidx2[j // 8, pl.ds((j % 8) * 16, 16)] = v
      return carry

    lax.fori_loop(0, KPT // 16, adjk, 0)
    pend = [
        pltpu.async_copy(skey_hbm.at[gidx2.at[g]],
                         gkeyv.at[pl.ds(g * 128, 128)], sem)
        for g in range(KPT // 128)
    ]
    for dsc in pend:
      dsc.wait()

    def scale(jj, carry):
      for u in range(2):
        j = jj * 2 + u
        kv = plsc.load_gather(gkeyv, [jnp.full((16,), j, jnp.int32)])
        sc = plsc.bitcast(jnp.full((16,), SENT, jnp.int32) - kv, jnp.float32)
        for q in range(8):
          rowsv[j, pl.ds(q * 16, 16)] = rowsv[j, pl.ds(q * 16, 16)] * sc
      return carry

    lax.fori_loop(0, KPT // 2, scale, 0)
    pltpu.sync_copy(rowsv, out_hbm.at[pl.ds((boff + b) * K + t * KPT, KPT)])


_SC_SCRATCH = [
        pltpu.VMEM_SHARED((NPAD,), jnp.int32),
        pltpu.VMEM_SHARED((NPAD,), jnp.int32),
        pltpu.VMEM_SHARED((NPAD,), jnp.int32),
        pltpu.VMEM_SHARED((NPAD,), jnp.int32),
        pltpu.VMEM_SHARED((NT * RADIX,), jnp.int32),
        pltpu.VMEM_SHARED((NT * RADIX,), jnp.int32),
        pltpu.VMEM_SHARED((NT * 16,), jnp.int32),
        pltpu.VMEM((CHUNK,), jnp.int32),
        pltpu.VMEM((CHUNK,), jnp.int32),
        pltpu.VMEM((CHUNK,), jnp.int32),
        pltpu.VMEM((NSCAT, 128), jnp.int32),
        pltpu.VMEM((RADIX,), jnp.int32),
        pltpu.VMEM((RADIX,), jnp.int32),
        pltpu.VMEM((DGT * NT,), jnp.int32),
        pltpu.VMEM((RADIX // 128, 128), jnp.int32),
        pltpu.VMEM((DGT * NT // 128, 128), jnp.int32),
        pltpu.VMEM((NT * 16,), jnp.int32),
        pltpu.VMEM((16,), jnp.int32),
        pltpu.VMEM((16,), jnp.int32),
        pltpu.VMEM((16,), jnp.int32),
        pltpu.VMEM((KPT,), jnp.int32),
        pltpu.VMEM((KPT,), jnp.int32),
        pltpu.VMEM((KPT // 128, 128), jnp.int32),
        pltpu.VMEM((KPT, D), jnp.float32),
        pltpu.SemaphoreType.DMA,
]


def _make_sc_topk(boff):
  @functools.partial(
      pl.kernel,
      out_type=(),
      mesh=plsc.VectorSubcoreMesh(core_axis_name="c", subcore_axis_name="s",
                                  num_cores=NC),
      compiler_params=pltpu.CompilerParams(needs_layout_passes=False),
      scratch_types=_SC_SCRATCH,
  )
  def _sc_topk(skey_hbm, h_hbm, out_hbm, *rest):
    _sc_body(boff, skey_hbm, h_hbm, out_hbm, *rest)

  return _sc_topk


_sc_topk01 = _make_sc_topk(0)
_sc_topk23 = _make_sc_topk(2)


def kernel(h, section_feature):
  h2d = h.reshape(B * N, D)
  pad = jnp.full((2, NPAD - N), SENT, jnp.int32)
  s01 = jnp.concatenate([_scores(h, section_feature, 0), pad], axis=1)
  s23 = jnp.concatenate([_scores(h, section_feature, 2), pad], axis=1)
  out_ref = jax.new_ref(jnp.zeros((B * K, D), jnp.float32))
  _sc_topk01(s01.reshape(2 * NPAD), h2d, out_ref)
  _sc_topk23(s23.reshape(2 * NPAD), h2d, out_ref)
  return out_ref[...].reshape(B, K, D)


# speedup vs baseline: 3.3581x; 1.0875x over previous
import functools

import jax
import jax.numpy as jnp
from jax import lax
from jax.experimental import pallas as pl
from jax.experimental.pallas import tpu as pltpu
from jax.experimental.pallas import tpu_sc as plsc

B = 4
N = 50000
D = 128
K = 8192

NT = 16
NC = 2
NPAD = 51200
CHUNK = NPAD // NT
NVREG = CHUNK // 16
RADIX = 1024
NPASS = 3
DGT = RADIX // NT
NSCAT = CHUNK // 128
KPT = K // NT
SENT = 0x3FFFFFFF

BLKN = 10000


def _score_body(h_ref, sf_ref, out_ref):
  w = lax.dot_general(
      sf_ref[0], h_ref[0],
      dimension_numbers=(((1,), (1,)), ((), ())),
      preferred_element_type=jnp.float32,
  )
  score = pl.reciprocal(1.0 + jnp.exp(-w), approx=False)
  skey = SENT - lax.bitcast_convert_type(score, jnp.int32)
  out_ref[0] = skey


def _scores(h, section_feature, boff):
  grid = (2, N // BLKN)
  out = pl.pallas_call(
      _score_body,
      grid=grid,
      in_specs=[
          pl.BlockSpec((1, BLKN, D), lambda b, j: (b + boff, j, 0)),
          pl.BlockSpec((1, 1, D), lambda b, j: (b + boff, 0, 0)),
      ],
      out_specs=pl.BlockSpec((1, 1, BLKN),
                             lambda b, j: (b * (N // BLKN) + j, 0, 0)),
      out_shape=jax.ShapeDtypeStruct((2 * (N // BLKN), 1, BLKN), jnp.int32),
  )(h, section_feature)
  return out.reshape(2, N)


def _sc_body(boff, skey_hbm, h_hbm, out_hbm,
             akey, aidx, bkey, bidx, grid_sh, base_sh, btot_sh,
             keyv, idxv, lrankv, posv, histv, basev, scanv,
             pubix2, posb2, btotv,
             tmpa, tmpb, tmpc, gkeyv, gidxv, gidx2, gkix2, rowsv, sem):
  c = lax.axis_index("c")
  t = lax.axis_index("s")
  iota16 = lax.iota(jnp.int32, 16)
  lane0 = iota16 == 0
  lane15 = iota16 == 15
  idxm1 = jnp.maximum(iota16 - 1, 0)
  idxp1 = jnp.minimum(iota16 + 1, 15)
  splat15 = jnp.full((16,), 15, jnp.int32)
  splat_t = jnp.full((16,), t, jnp.int32)
  iota_r = iota16 * RADIX
  zeros16 = jnp.zeros((16,), jnp.int32)

  def initix(j, carry):
    pubix2[j // 8, pl.ds((j % 8) * 16, 16)] = (j * 16 + iota16) * NT + t
    return carry

  lax.fori_loop(0, RADIX // 16, initix, 0)

  def initpb(j, carry):
    posb2[j // 8, pl.ds((j % 8) * 16, 16)] = iota16 * RADIX + t * DGT + j
    return carry

  lax.fori_loop(0, DGT * NT // 16, initpb, 0)

  if True:
    b = c
    for p in range(NPASS):
      shift = 10 * p
      if p % 2 == 0:
        dst_key, dst_idx = akey, aidx
        src_key, src_idx = bkey, bidx
      else:
        dst_key, dst_idx = bkey, bidx
        src_key, src_idx = akey, aidx
      if p == 0:
        pltpu.sync_copy(skey_hbm.at[pl.ds(b * NPAD + t * CHUNK, CHUNK)], keyv)

        def genidx(j, carry):
          idxv[pl.ds(j * 16, 16)] = t * CHUNK + j * 16 + iota16
          return carry

        lax.fori_loop(0, NVREG, genidx, 0)
      else:
        pltpu.sync_copy(src_key.at[pl.ds(t * CHUNK, CHUNK)], keyv)
        pltpu.sync_copy(src_idx.at[pl.ds(t * CHUNK, CHUNK)], idxv)
      for z in range(RADIX // 16):
        histv[pl.ds(z * 16, 16)] = zeros16

      def step_a(jj, carry):
        for u in range(4):
          j = jj * 4 + u
          k = keyv[pl.ds(j * 16, 16)]
          d = (k >> shift) & (RADIX - 1)
          ds, lanes = plsc.sort_key_val(d, iota16)
          inv = plsc.sort_key_val(lanes, iota16)[1]
          prev = jnp.take_along_axis(ds, idxm1, axis=0, mode=lax.GatherScatterMode.PROMISE_IN_BOUNDS)
          is_start = lane0 | (ds != prev)
          startpos = plsc.cummax(jnp.where(is_start, iota16, 0))
          runrank = iota16 - startpos
          cur = plsc.load_gather(histv, [ds])
          lrank_s = cur + runrank
          st = jnp.where(is_start, 1, 0)
          nxt = jnp.take_along_axis(st, idxp1, axis=0, mode=lax.GatherScatterMode.PROMISE_IN_BOUNDS)
          is_last = lane15 | (nxt == 1)
          plsc.store_scatter(histv, [ds], lrank_s + 1, mask=is_last)
          lrankv[pl.ds(j * 16, 16)] = jnp.take_along_axis(
              lrank_s, inv, axis=0,
              mode=lax.GatherScatterMode.PROMISE_IN_BOUNDS)
        return carry

      lax.fori_loop(0, NVREG // 4, step_a, 0)
      pend = [pltpu.async_copy(histv.at[pl.ds(g * 128, 128)],
                               grid_sh.at[pubix2.at[g]], sem)
              for g in range(RADIX // 128)]
      for dsc in pend:
        dsc.wait()
      plsc.subcore_barrier()
      pltpu.sync_copy(grid_sh.at[pl.ds(t * DGT * NT, DGT * NT)], scanv)

      def step_s(dl, run):
        cvec = scanv[pl.ds(dl * 16, 16)]
        inc = plsc.cumsum(cvec)
        tmpb[pl.ds(0, 16)] = inc
        tot = plsc.load_gather(tmpb, [splat15])
        scanv[pl.ds(dl * 16, 16)] = run + (inc - cvec)
        return run + tot

      blocktot = lax.fori_loop(0, DGT, step_s, zeros16)
      tmpa[pl.ds(0, 16)] = blocktot
      pltpu.sync_copy(tmpa, btot_sh.at[pl.ds(t * 16, 16)])
      pend = [pltpu.async_copy(scanv.at[pl.ds(g * 128, 128)],
                               base_sh.at[posb2.at[g]], sem)
              for g in range(DGT * NT // 128)]
      for dsc in pend:
        dsc.wait()
      plsc.subcore_barrier()
      pltpu.sync_copy(btot_sh, btotv)
      bts = plsc.load_gather(btotv, [iota16 * 16])
      boffs = plsc.cumsum(bts) - bts
      tmpc[pl.ds(0, 16)] = boffs
      pltpu.sync_copy(base_sh.at[pl.ds(t * RADIX, RADIX)], basev)

      def addoff(j, carry):
        off = plsc.load_gather(tmpc, [jnp.full((16,), j // (DGT // 16),
                                               jnp.int32)])
        basev[pl.ds(j * 16, 16)] = basev[pl.ds(j * 16, 16)] + off
        return carry

      lax.fori_loop(0, RADIX // 16, addoff, 0)

      def step_b(jj, carry):
        for u in range(4):
          j = jj * 4 + u
          k = keyv[pl.ds(j * 16, 16)]
          d = (k >> shift) & (RADIX - 1)
          lr = lrankv[pl.ds(j * 16, 16)]
          pos = plsc.load_gather(basev, [d]) + lr
          posv[j // 8, pl.ds((j % 8) * 16, 16)] = pos
        return carry

      lax.fori_loop(0, NVREG // 4, step_b, 0)
      pend = []
      for q in range(NSCAT):
        if p != NPASS - 1:
          pend.append(pltpu.async_copy(keyv.at[pl.ds(q * 128, 128)],
                                       dst_key.at[posv.at[q]], sem))
        pend.append(pltpu.async_copy(idxv.at[pl.ds(q * 128, 128)],
                                     dst_idx.at[posv.at[q]], sem))
      for dsc in pend:
        dsc.wait()
      plsc.subcore_barrier()

    fin_idx = aidx if (NPASS - 1) % 2 == 0 else bidx
    pltpu.sync_copy(fin_idx.at[pl.ds(t * KPT, KPT)], gidxv)
    base_row = (boff + b) * N
    base_key = b * NPAD

    def adj(j, carry):
      v = gidxv[pl.ds(j * 16, 16)] + base_row
      gidx2[j // 8, pl.ds((j % 8) * 16, 16)] = v
      return carry

    lax.fori_loop(0, KPT // 16, adj, 0)

    def adjk(j, carry):
      v = gidxv[pl.ds(j * 16, 16)] + base_key
      gkix2[j // 8, pl.ds((j % 8) * 16, 16)] = v
      return carry

    lax.fori_loop(0, KPT // 16, adjk, 0)
    kpend = [
        pltpu.async_copy(skey_hbm.at[gkix2.at[g]],
                         gkeyv.at[pl.ds(g * 128, 128)], sem)
        for g in range(KPT // 128)
    ]
    rpend = [
        pltpu.async_copy(h_hbm.at[gidx2.at[g]],
                         rowsv.at[pl.ds(g * 128, 128)], sem)
        for g in range(KPT // 128)
    ]
    for dsc in kpend:
      dsc.wait()
    opend = []
    out_base = (boff + b) * K + t * KPT
    for g in range(KPT // 128):
      rpend[g].wait()

      def scale(jj, carry, g=g):
        for u in range(2):
          j = g * 128 + jj * 2 + u
          kv = plsc.load_gather(gkeyv, [jnp.full((16,), j, jnp.int32)])
          sc = plsc.bitcast(jnp.full((16,), SENT, jnp.int32) - kv,
                            jnp.float32)
          for q in range(8):
            rowsv[j, pl.ds(q * 16, 16)] = rowsv[j, pl.ds(q * 16, 16)] * sc
        return carry

      lax.fori_loop(0, 64, scale, 0)
      opend.append(pltpu.async_copy(
          rowsv.at[pl.ds(g * 128, 128)],
          out_hbm.at[pl.ds(out_base + g * 128, 128)], sem))
    for dsc in opend:
      dsc.wait()


_SC_SCRATCH = [
        pltpu.VMEM_SHARED((NPAD,), jnp.int32),
        pltpu.VMEM_SHARED((NPAD,), jnp.int32),
        pltpu.VMEM_SHARED((NPAD,), jnp.int32),
        pltpu.VMEM_SHARED((NPAD,), jnp.int32),
        pltpu.VMEM_SHARED((NT * RADIX,), jnp.int32),
        pltpu.VMEM_SHARED((NT * RADIX,), jnp.int32),
        pltpu.VMEM_SHARED((NT * 16,), jnp.int32),
        pltpu.VMEM((CHUNK,), jnp.int32),
        pltpu.VMEM((CHUNK,), jnp.int32),
        pltpu.VMEM((CHUNK,), jnp.int32),
        pltpu.VMEM((NSCAT, 128), jnp.int32),
        pltpu.VMEM((RADIX,), jnp.int32),
        pltpu.VMEM((RADIX,), jnp.int32),
        pltpu.VMEM((DGT * NT,), jnp.int32),
        pltpu.VMEM((RADIX // 128, 128), jnp.int32),
        pltpu.VMEM((DGT * NT // 128, 128), jnp.int32),
        pltpu.VMEM((NT * 16,), jnp.int32),
        pltpu.VMEM((16,), jnp.int32),
        pltpu.VMEM((16,), jnp.int32),
        pltpu.VMEM((16,), jnp.int32),
        pltpu.VMEM((KPT,), jnp.int32),
        pltpu.VMEM((KPT,), jnp.int32),
        pltpu.VMEM((KPT // 128, 128), jnp.int32),
        pltpu.VMEM((KPT // 128, 128), jnp.int32),
        pltpu.VMEM((KPT, D), jnp.float32),
        pltpu.SemaphoreType.DMA,
]


def _make_sc_topk(boff):
  @functools.partial(
      pl.kernel,
      out_type=(),
      mesh=plsc.VectorSubcoreMesh(core_axis_name="c", subcore_axis_name="s",
                                  num_cores=NC),
      compiler_params=pltpu.CompilerParams(needs_layout_passes=False),
      scratch_types=_SC_SCRATCH,
  )
  def _sc_topk(skey_hbm, h_hbm, out_hbm, *rest):
    _sc_body(boff, skey_hbm, h_hbm, out_hbm, *rest)

  return _sc_topk


_sc_topk01 = _make_sc_topk(0)
_sc_topk23 = _make_sc_topk(2)


def kernel(h, section_feature):
  h2d = h.reshape(B * N, D)
  pad = jnp.full((2, NPAD - N), SENT, jnp.int32)
  s01 = jnp.concatenate([_scores(h, section_feature, 0), pad], axis=1)
  s23 = jnp.concatenate([_scores(h, section_feature, 2), pad], axis=1)
  out_ref = jax.new_ref(jnp.zeros((B * K, D), jnp.float32))
  _sc_topk01(s01.reshape(2 * NPAD), h2d, out_ref)
  _sc_topk23(s23.reshape(2 * NPAD), h2d, out_ref)
  return out_ref[...].reshape(B, K, D)


# speedup vs baseline: 3.3664x; 1.0025x over previous
import functools

import jax
import jax.numpy as jnp
from jax import lax
from jax.experimental import pallas as pl
from jax.experimental.pallas import tpu as pltpu
from jax.experimental.pallas import tpu_sc as plsc

B = 4
N = 50000
D = 128
K = 8192

NT = 16
NC = 2
NPAD = 51200
CHUNK = NPAD // NT
NVREG = CHUNK // 16
RADIX = 1024
NPASS = 3
DGT = RADIX // NT
NSCAT = CHUNK // 128
KPT = K // NT
SENT = 0x3FFFFFFF

BLKN = 10000


def _score_body(h_ref, sf_ref, out_ref):
  w = lax.dot_general(
      sf_ref[0], h_ref[0],
      dimension_numbers=(((1,), (1,)), ((), ())),
      preferred_element_type=jnp.float32,
  )
  score = pl.reciprocal(1.0 + jnp.exp(-w), approx=False)
  skey = SENT - lax.bitcast_convert_type(score, jnp.int32)
  out_ref[0] = skey


def _scores(h, section_feature, boff):
  grid = (2, N // BLKN)
  out = pl.pallas_call(
      _score_body,
      grid=grid,
      in_specs=[
          pl.BlockSpec((1, BLKN, D), lambda b, j: (b + boff, j, 0)),
          pl.BlockSpec((1, 1, D), lambda b, j: (b + boff, 0, 0)),
      ],
      out_specs=pl.BlockSpec((1, 1, BLKN),
                             lambda b, j: (b * (N // BLKN) + j, 0, 0)),
      out_shape=jax.ShapeDtypeStruct((2 * (N // BLKN), 1, BLKN), jnp.int32),
  )(h, section_feature)
  return out.reshape(2, N)


def _sc_body(boff, skey_hbm, h_hbm, out_hbm,
             akey, aidx, bkey, bidx, grid_sh, base_sh, btot_sh,
             keyv, idxv, lrankv, posv, histv, basev, scanv,
             pubix2, posb2, btotv,
             tmpa, tmpb, tmpc, gkeyv, gidxv, gidx2, gkix2, rowsv, sem):
  c = lax.axis_index("c")
  t = lax.axis_index("s")
  iota16 = lax.iota(jnp.int32, 16)
  lane0 = iota16 == 0
  lane15 = iota16 == 15
  idxm1 = jnp.maximum(iota16 - 1, 0)
  idxp1 = jnp.minimum(iota16 + 1, 15)
  splat15 = jnp.full((16,), 15, jnp.int32)
  splat_t = jnp.full((16,), t, jnp.int32)
  iota_r = iota16 * RADIX
  zeros16 = jnp.zeros((16,), jnp.int32)

  def initix(j, carry):
    pubix2[j // 8, pl.ds((j % 8) * 16, 16)] = (j * 16 + iota16) * NT + t
    return carry

  lax.fori_loop(0, RADIX // 16, initix, 0)

  def initpb(j, carry):
    posb2[j // 8, pl.ds((j % 8) * 16, 16)] = iota16 * RADIX + t * DGT + j
    return carry

  lax.fori_loop(0, DGT * NT // 16, initpb, 0)

  if True:
    b = c
    for p in range(NPASS):
      shift = 10 * p
      if p % 2 == 0:
        dst_key, dst_idx = akey, aidx
        src_key, src_idx = bkey, bidx
      else:
        dst_key, dst_idx = bkey, bidx
        src_key, src_idx = akey, aidx
      if p == 0:
        pltpu.sync_copy(skey_hbm.at[pl.ds(b * NPAD + t * CHUNK, CHUNK)], keyv)

        def genidx(j, carry):
          idxv[pl.ds(j * 16, 16)] = t * CHUNK + j * 16 + iota16
          return carry

        lax.fori_loop(0, NVREG, genidx, 0)
      else:
        d1 = pltpu.async_copy(src_key.at[pl.ds(t * CHUNK, CHUNK)], keyv, sem)
        d2 = pltpu.async_copy(src_idx.at[pl.ds(t * CHUNK, CHUNK)], idxv, sem)
        d1.wait()
        d2.wait()
      for z in range(RADIX // 16):
        histv[pl.ds(z * 16, 16)] = zeros16

      def step_a(jj, carry):
        for u in range(4):
          j = jj * 4 + u
          k = keyv[pl.ds(j * 16, 16)]
          d = (k >> shift) & (RADIX - 1)
          ds, lanes = plsc.sort_key_val(d, iota16)
          inv = plsc.sort_key_val(lanes, iota16)[1]
          prev = jnp.take_along_axis(ds, idxm1, axis=0, mode=lax.GatherScatterMode.PROMISE_IN_BOUNDS)
          is_start = lane0 | (ds != prev)
          startpos = plsc.cummax(jnp.where(is_start, iota16, 0))
          runrank = iota16 - startpos
          cur = plsc.load_gather(histv, [ds])
          lrank_s = cur + runrank
          st = jnp.where(is_start, 1, 0)
          nxt = jnp.take_along_axis(st, idxp1, axis=0, mode=lax.GatherScatterMode.PROMISE_IN_BOUNDS)
          is_last = lane15 | (nxt == 1)
          plsc.store_scatter(histv, [ds], lrank_s + 1, mask=is_last)
          lrankv[pl.ds(j * 16, 16)] = jnp.take_along_axis(
              lrank_s, inv, axis=0,
              mode=lax.GatherScatterMode.PROMISE_IN_BOUNDS)
        return carry

      lax.fori_loop(0, NVREG // 4, step_a, 0)
      pend = [pltpu.async_copy(histv.at[pl.ds(g * 128, 128)],
                               grid_sh.at[pubix2.at[g]], sem)
              for g in range(RADIX // 128)]
      for dsc in pend:
        dsc.wait()
      plsc.subcore_barrier()
      pltpu.sync_copy(grid_sh.at[pl.ds(t * DGT * NT, DGT * NT)], scanv)

      def step_s(dl, run):
        cvec = scanv[pl.ds(dl * 16, 16)]
        inc = plsc.cumsum(cvec)
        tmpb[pl.ds(0, 16)] = inc
        tot = plsc.load_gather(tmpb, [splat15])
        scanv[pl.ds(dl * 16, 16)] = run + (inc - cvec)
        return run + tot

      blocktot = lax.fori_loop(0, DGT, step_s, zeros16)
      tmpa[pl.ds(0, 16)] = blocktot
      pltpu.sync_copy(tmpa, btot_sh.at[pl.ds(t * 16, 16)])
      pend = [pltpu.async_copy(scanv.at[pl.ds(g * 128, 128)],
                               base_sh.at[posb2.at[g]], sem)
              for g in range(DGT * NT // 128)]
      for dsc in pend:
        dsc.wait()
      plsc.subcore_barrier()
      pltpu.sync_copy(btot_sh, btotv)
      bts = plsc.load_gather(btotv, [iota16 * 16])
      boffs = plsc.cumsum(bts) - bts
      tmpc[pl.ds(0, 16)] = boffs
      pltpu.sync_copy(base_sh.at[pl.ds(t * RADIX, RADIX)], basev)

      def addoff(j, carry):
        off = plsc.load_gather(tmpc, [jnp.full((16,), j // (DGT // 16),
                                               jnp.int32)])
        basev[pl.ds(j * 16, 16)] = basev[pl.ds(j * 16, 16)] + off
        return carry

      lax.fori_loop(0, RADIX // 16, addoff, 0)

      def step_b(jj, carry):
        for u in range(4):
          j = jj * 4 + u
          k = keyv[pl.ds(j * 16, 16)]
          d = (k >> shift) & (RADIX - 1)
          lr = lrankv[pl.ds(j * 16, 16)]
          pos = plsc.load_gather(basev, [d]) + lr
          posv[j // 8, pl.ds((j % 8) * 16, 16)] = pos
        return carry

      lax.fori_loop(0, NVREG // 4, step_b, 0)
      pend = []
      for q in range(NSCAT):
        if p != NPASS - 1:
          pend.append(pltpu.async_copy(keyv.at[pl.ds(q * 128, 128)],
                                       dst_key.at[posv.at[q]], sem))
        pend.append(pltpu.async_copy(idxv.at[pl.ds(q * 128, 128)],
                                     dst_idx.at[posv.at[q]], sem))
      for dsc in pend:
        dsc.wait()
      plsc.subcore_barrier()

    fin_idx = aidx if (NPASS - 1) % 2 == 0 else bidx
    pltpu.sync_copy(fin_idx.at[pl.ds(t * KPT, KPT)], gidxv)
    base_row = (boff + b) * N
    base_key = b * NPAD

    def adj(j, carry):
      v = gidxv[pl.ds(j * 16, 16)] + base_row
      gidx2[j // 8, pl.ds((j % 8) * 16, 16)] = v
      return carry

    lax.fori_loop(0, KPT // 16, adj, 0)

    def adjk(j, carry):
      v = gidxv[pl.ds(j * 16, 16)] + base_key
      gkix2[j // 8, pl.ds((j % 8) * 16, 16)] = v
      return carry

    lax.fori_loop(0, KPT // 16, adjk, 0)
    kpend = [
        pltpu.async_copy(skey_hbm.at[gkix2.at[g]],
                         gkeyv.at[pl.ds(g * 128, 128)], sem)
        for g in range(KPT // 128)
    ]
    rpend = [
        pltpu.async_copy(h_hbm.at[gidx2.at[g]],
                         rowsv.at[pl.ds(g * 128, 128)], sem)
        for g in range(KPT // 128)
    ]
    for dsc in kpend:
      dsc.wait()
    opend = []
    out_base = (boff + b) * K + t * KPT
    for g in range(KPT // 128):
      rpend[g].wait()

      def scale(jj, carry, g=g):
        for u in range(2):
          j = g * 128 + jj * 2 + u
          kv = plsc.load_gather(gkeyv, [jnp.full((16,), j, jnp.int32)])
          sc = plsc.bitcast(jnp.full((16,), SENT, jnp.int32) - kv,
                            jnp.float32)
          for q in range(8):
            rowsv[j, pl.ds(q * 16, 16)] = rowsv[j, pl.ds(q * 16, 16)] * sc
        return carry

      lax.fori_loop(0, 64, scale, 0)
      opend.append(pltpu.async_copy(
          rowsv.at[pl.ds(g * 128, 128)],
          out_hbm.at[pl.ds(out_base + g * 128, 128)], sem))
    for dsc in opend:
      dsc.wait()


_SC_SCRATCH = [
        pltpu.VMEM_SHARED((NPAD,), jnp.int32),
        pltpu.VMEM_SHARED((NPAD,), jnp.int32),
        pltpu.VMEM_SHARED((NPAD,), jnp.int32),
        pltpu.VMEM_SHARED((NPAD,), jnp.int32),
        pltpu.VMEM_SHARED((NT * RADIX,), jnp.int32),
        pltpu.VMEM_SHARED((NT * RADIX,), jnp.int32),
        pltpu.VMEM_SHARED((NT * 16,), jnp.int32),
        pltpu.VMEM((CHUNK,), jnp.int32),
        pltpu.VMEM((CHUNK,), jnp.int32),
        pltpu.VMEM((CHUNK,), jnp.int32),
        pltpu.VMEM((NSCAT, 128), jnp.int32),
        pltpu.VMEM((RADIX,), jnp.int32),
        pltpu.VMEM((RADIX,), jnp.int32),
        pltpu.VMEM((DGT * NT,), jnp.int32),
        pltpu.VMEM((RADIX // 128, 128), jnp.int32),
        pltpu.VMEM((DGT * NT // 128, 128), jnp.int32),
        pltpu.VMEM((NT * 16,), jnp.int32),
        pltpu.VMEM((16,), jnp.int32),
        pltpu.VMEM((16,), jnp.int32),
        pltpu.VMEM((16,), jnp.int32),
        pltpu.VMEM((KPT,), jnp.int32),
        pltpu.VMEM((KPT,), jnp.int32),
        pltpu.VMEM((KPT // 128, 128), jnp.int32),
        pltpu.VMEM((KPT // 128, 128), jnp.int32),
        pltpu.VMEM((KPT, D), jnp.float32),
        pltpu.SemaphoreType.DMA,
]


def _make_sc_topk(boff):
  @functools.partial(
      pl.kernel,
      out_type=(),
      mesh=plsc.VectorSubcoreMesh(core_axis_name="c", subcore_axis_name="s",
                                  num_cores=NC),
      compiler_params=pltpu.CompilerParams(needs_layout_passes=False),
      scratch_types=_SC_SCRATCH,
  )
  def _sc_topk(skey_hbm, h_hbm, out_hbm, *rest):
    _sc_body(boff, skey_hbm, h_hbm, out_hbm, *rest)

  return _sc_topk


_sc_topk01 = _make_sc_topk(0)
_sc_topk23 = _make_sc_topk(2)


def kernel(h, section_feature):
  h2d = h.reshape(B * N, D)
  pad = jnp.full((2, NPAD - N), SENT, jnp.int32)
  s01 = jnp.concatenate([_scores(h, section_feature, 0), pad], axis=1)
  s23 = jnp.concatenate([_scores(h, section_feature, 2), pad], axis=1)
  out_ref = jax.new_ref(jnp.zeros((B * K, D), jnp.float32))
  _sc_topk01(s01.reshape(2 * NPAD), h2d, out_ref)
  _sc_topk23(s23.reshape(2 * NPAD), h2d, out_ref)
  return out_ref[...].reshape(B, K, D)
